# baseline (speedup 1.0000x reference)
import functools

import jax
import jax.numpy as jnp
from jax import lax
from jax.experimental import pallas as pl
from jax.experimental.pallas import tpu as pltpu
from jax.experimental.pallas import tpu_sc as plsc

NUM_WORKERS = 32
NUM_CORES = 2
LANE = 128
SUB = 8


@jax.jit
def _detile(tT, tail):
    d, v = tT.shape
    ngrp = (v + 127) // 128
    full = v // 128
    rem = v - full * 128
    dgs = d // SUB
    words = 128 * d
    per_w = (full + NUM_WORKERS - 1) // NUM_WORKERS
    n2 = (per_w + (per_w & 1)) // 2

    mesh = plsc.VectorSubcoreMesh(core_axis_name="c", subcore_axis_name="s")

    @functools.partial(
        pl.kernel,
        out_type=jax.ShapeDtypeStruct((v * d,), jnp.float32),
        mesh=mesh,
        scratch_types=[
            pltpu.VMEM((d, 128), jnp.float32),
            pltpu.VMEM((d, 128), jnp.float32),
            pltpu.VMEM((words,), jnp.float32),
            pltpu.VMEM((words,), jnp.float32),
            pltpu.SemaphoreType.DMA,
            pltpu.SemaphoreType.DMA,
            pltpu.SemaphoreType.DMA,
            pltpu.SemaphoreType.DMA,
        ],
        compiler_params=pltpu.CompilerParams(
            use_tc_tiling_on_sc=True, needs_layout_passes=False
        ),
    )
    def k(tT_hbm, tail_hbm, out_hbm, in0, in1, ob0, ob1, si0, si1, so0, so1):
        wid = lax.axis_index("s") * NUM_CORES + lax.axis_index("c")
        inb = (in0, in1)
        outb = (ob0, ob1)
        si = (si0, si1)
        so = (so0, so1)

        iota = lax.iota(jnp.int32, 16)

        def cidx(i):
            return jnp.minimum(wid * per_w + i, full - 1)

        def fire_in(i, p):
            c = cidx(i)
            for dg in range(dgs):
                pltpu.async_copy(
                    tT_hbm.at[pl.ds(SUB * dg, SUB), pl.ds(128 * c, 128)],
                    inb[p].at[pl.ds(SUB * dg, SUB)],
                    si[p],
                )

        def wait_in(p):
            for dg in range(dgs):
                pltpu.make_async_copy(
                    tT_hbm.at[pl.ds(0, SUB), pl.ds(0, 128)],
                    inb[p].at[pl.ds(SUB * dg, SUB)],
                    si[p],
                ).wait()

        def fire_out(i, p):
            pltpu.async_copy(outb[p], out_hbm.at[pl.ds(words * cidx(i), words)], so[p])

        def wait_out(i, p):
            pltpu.make_async_copy(
                outb[p], out_hbm.at[pl.ds(words * cidx(i), words)], so[p]
            ).wait()

        def transpose(p, lgroups):
            ip, op = inb[p], outb[p]

            def tbody(sv, carry):
                for su in range(4):
                    s = 4 * sv + su
                    lv0 = (iota + s) & 15
                    lvd = lv0 * d + iota
                    for dg0 in range(dgs // 2):
                        base = iota + 16 * dg0 if dg0 else iota
                        dst0 = lvd + 16 * dg0 if dg0 else lvd
                        for l0 in range(0, 16 * lgroups, 16):
                            lv = lv0 + l0 if l0 else lv0
                            dst = dst0 + l0 * d if l0 else dst0
                            val = plsc.load_gather(ip, [base, lv])
                            plsc.store_scatter(op, [dst], val)
                return carry

            lax.fori_loop(0, 4, tbody, 0)

        fire_in(0, 0)
        fire_in(1, 1)

        def body(i2, carry):
            for p in range(2):
                i = 2 * i2 + p
                wait_in(p)

                @pl.when(i2 > 0)
                def _():
                    wait_out(i - 2, p)

                transpose(p, 8)
                fire_out(i, p)

                @pl.when(i + 2 < 2 * n2)
                def _():
                    fire_in(i + 2, p)
            return carry

        lax.fori_loop(0, n2, body, 0)
        wait_out(2 * n2 - 2, 0)
        wait_out(2 * n2 - 1, 1)

        if rem:
            @pl.when(wid == NUM_WORKERS - 1)
            def _():
                pltpu.sync_copy(tail_hbm, outb[0].at[pl.ds(0, rem * d)])
                pltpu.sync_copy(
                    outb[0].at[pl.ds(0, rem * d)],
                    out_hbm.at[pl.ds(words * full, rem * d)],
                )

    return k(tT, tail)


@functools.partial(jax.jit, static_argnames=("hist",))
def _embed(table, ids_raw, hist):
    d = table.shape[1]
    hgs, bbs = ids_raw.shape[0], ids_raw.shape[1]
    dgs = d // SUB
    h2s = hist // 2

    mesh = plsc.VectorSubcoreMesh(core_axis_name="c", subcore_axis_name="s")

    @functools.partial(
        pl.kernel,
        out_type=jax.ShapeDtypeStruct((hist, dgs, bbs, SUB * LANE), jnp.float32),
        mesh=mesh,
        scratch_types=[
            pltpu.VMEM((hgs, SUB, LANE), jnp.int32),
            pltpu.VMEM((LANE, d), jnp.float32),
            pltpu.VMEM((LANE, d), jnp.float32),
            pltpu.VMEM((LANE, d), jnp.float32),
            pltpu.VMEM((LANE, d), jnp.float32),
            pltpu.VMEM((d * LANE,), jnp.float32),
            pltpu.VMEM((d * LANE,), jnp.float32),
            pltpu.SemaphoreType.DMA,
            pltpu.SemaphoreType.DMA,
            pltpu.SemaphoreType.DMA,
            pltpu.SemaphoreType.DMA,
            pltpu.SemaphoreType.DMA,
            pltpu.SemaphoreType.DMA,
        ],
        compiler_params=pltpu.CompilerParams(use_tc_tiling_on_sc=False, needs_layout_passes=False),
    )
    def k(table_hbm, ids_hbm, out_hbm, idx_v, rows0, rows1, rows2, rows3,
          t0, t1, sg0, sg1, sg2, sg3, ss0, ss1):
        wid = lax.axis_index("s") * NUM_CORES + lax.axis_index("c")
        rows = (rows0, rows1, rows2, rows3)
        tiles = (t0, t1)
        sg = (sg0, sg1, sg2, sg3)
        ss = (ss0, ss1)

        for hg in range(hgs):
            pltpu.sync_copy(ids_hbm.at[hg, wid], idx_v.at[hg])

        iota = lax.iota(jnp.int32, 16)

        def fire_gather(h, p):
            pltpu.async_copy(
                table_hbm.at[idx_v.at[h >> 3, h & 7]], rows[p], sg[p]
            )

        def wait_gather(p):
            pltpu.make_async_copy(table_hbm.at[idx_v.at[0, 0]], rows[p], sg[p]).wait()

        def fire_stores(h, p):
            for dg in range(dgs):
                pltpu.async_copy(
                    tiles[p].at[pl.ds(SUB * LANE * dg, SUB * LANE)],
                    out_hbm.at[h, dg, wid],
                    ss[p],
                )

        def wait_stores(h, p):
            for dg in range(dgs):
                pltpu.make_async_copy(
                    tiles[p].at[pl.ds(SUB * LANE * dg, SUB * LANE)],
                    out_hbm.at[h, dg, wid],
                    ss[p],
                ).wait()

        for q in range(4):
            fire_gather(q, q)

        def body(h4, carry):
            for q in range(4):
                h = 4 * h4 + q
                p = q & 1
                wait_gather(q)

                if q < 2:
                    @pl.when(h4 > 0)
                    def _():
                        wait_stores(h - 2, p)
                else:
                    wait_stores(h - 2, p)

                rp, tp = rows[q], tiles[p]

                def tbody(sv, c):
                    for su in range(4):
                        s = 4 * sv + su
                        dvec = (iota + s) & 15
                        dsti = dvec * LANE + iota
                        for d0 in range(0, d, 16):
                            dv = dvec + d0 if d0 else dvec
                            for lg in range(LANE // 16):
                                bv = iota + 16 * lg if lg else iota
                                dst = dsti + (d0 * LANE + 16 * lg) if d0 or lg else dsti
                                v = plsc.load_gather(rp, [bv, dv])
                                plsc.store_scatter(tp, [dst], v)
                    return c

                lax.fori_loop(0, 4, tbody, 0)
                fire_stores(h, p)

                @pl.when(h + 4 < hist)
                def _():
                    fire_gather(h + 4, q)
            return carry

        lax.fori_loop(0, hist // 4, body, 0)
        wait_stores(hist - 2, 0)
        wait_stores(hist - 1, 1)

    return k(table, ids_raw)


def kernel(input_ids, embedding_table):
    bsz, hist = input_ids.shape
    d = embedding_table.shape[1]
    bbs, hgs = bsz // LANE, hist // SUB
    ids_raw = (
        input_ids.astype(jnp.int32)
        .reshape(bbs, LANE, hgs, SUB)
        .transpose(2, 0, 3, 1)
    )
    vocab = embedding_table.shape[0]
    rem = vocab % LANE
    tail = embedding_table[vocab - rem:].reshape(-1)
    table_rm = _detile(embedding_table.T, tail).reshape(embedding_table.shape)
    out4d = _embed(table_rm, ids_raw, hist)
    out5d = out4d.reshape(hist, d // SUB, bbs, SUB, LANE)
    return out5d.transpose(2, 4, 0, 1, 3).reshape(bsz, hist, d)


# speedup vs baseline: 1.1800x; 1.1800x over previous
import functools

import jax
import jax.numpy as jnp
from jax import lax
from jax.experimental import pallas as pl
from jax.experimental.pallas import tpu as pltpu
from jax.experimental.pallas import tpu_sc as plsc

NUM_WORKERS = 32
NUM_CORES = 2
LANE = 128
SUB = 8


@jax.jit
def _detile(tT, tail):
    d, v = tT.shape
    ngrp = (v + 127) // 128
    full = v // 128
    rem = v - full * 128
    dgs = d // SUB
    words = 128 * d
    per_w = (full + NUM_WORKERS - 1) // NUM_WORKERS
    n2 = (per_w + (per_w & 1)) // 2

    mesh = plsc.VectorSubcoreMesh(core_axis_name="c", subcore_axis_name="s")

    @functools.partial(
        pl.kernel,
        out_type=jax.ShapeDtypeStruct((v * d,), jnp.float32),
        mesh=mesh,
        scratch_types=[
            pltpu.VMEM((d, 128), jnp.float32),
            pltpu.VMEM((d, 128), jnp.float32),
            pltpu.VMEM((words,), jnp.float32),
            pltpu.VMEM((words,), jnp.float32),
            pltpu.SemaphoreType.DMA,
            pltpu.SemaphoreType.DMA,
            pltpu.SemaphoreType.DMA,
            pltpu.SemaphoreType.DMA,
        ],
        compiler_params=pltpu.CompilerParams(
            use_tc_tiling_on_sc=True, needs_layout_passes=False
        ),
    )
    def k(tT_hbm, tail_hbm, out_hbm, in0, in1, ob0, ob1, si0, si1, so0, so1):
        wid = lax.axis_index("s") * NUM_CORES + lax.axis_index("c")
        inb = (in0, in1)
        outb = (ob0, ob1)
        si = (si0, si1)
        so = (so0, so1)

        iota = lax.iota(jnp.int32, 16)

        def cidx(i):
            return jnp.minimum(wid * per_w + i, full - 1)

        def fire_in(i, p):
            c = cidx(i)
            for dg in range(dgs):
                pltpu.async_copy(
                    tT_hbm.at[pl.ds(SUB * dg, SUB), pl.ds(128 * c, 128)],
                    inb[p].at[pl.ds(SUB * dg, SUB)],
                    si[p],
                )

        def wait_in(p):
            for dg in range(dgs):
                pltpu.make_async_copy(
                    tT_hbm.at[pl.ds(0, SUB), pl.ds(0, 128)],
                    inb[p].at[pl.ds(SUB * dg, SUB)],
                    si[p],
                ).wait()

        def fire_out(i, p):
            pltpu.async_copy(outb[p], out_hbm.at[pl.ds(words * cidx(i), words)], so[p])

        def wait_out(i, p):
            pltpu.make_async_copy(
                outb[p], out_hbm.at[pl.ds(words * cidx(i), words)], so[p]
            ).wait()

        def transpose(p, lgroups):
            ip, op = inb[p], outb[p]

            def tbody(sv, carry):
                for su in range(4):
                    s = 4 * sv + su
                    lv0 = (iota + s) & 15
                    lvd = lv0 * d + iota
                    for dg0 in range(dgs // 2):
                        base = iota + 16 * dg0 if dg0 else iota
                        dst0 = lvd + 16 * dg0 if dg0 else lvd
                        for l0 in range(0, 16 * lgroups, 16):
                            lv = lv0 + l0 if l0 else lv0
                            dst = dst0 + l0 * d if l0 else dst0
                            val = plsc.load_gather(ip, [base, lv])
                            plsc.store_scatter(op, [dst], val)
                return carry

            lax.fori_loop(0, 4, tbody, 0)

        fire_in(0, 0)
        fire_in(1, 1)

        def body(i2, carry):
            for p in range(2):
                i = 2 * i2 + p
                wait_in(p)

                @pl.when(i2 > 0)
                def _():
                    wait_out(i - 2, p)

                transpose(p, 8)
                fire_out(i, p)

                @pl.when(i + 2 < 2 * n2)
                def _():
                    fire_in(i + 2, p)
            return carry

        lax.fori_loop(0, n2, body, 0)
        wait_out(2 * n2 - 2, 0)
        wait_out(2 * n2 - 1, 1)

        if rem:
            @pl.when(wid == NUM_WORKERS - 1)
            def _():
                pltpu.sync_copy(tail_hbm, outb[0].at[pl.ds(0, rem * d)])
                pltpu.sync_copy(
                    outb[0].at[pl.ds(0, rem * d)],
                    out_hbm.at[pl.ds(words * full, rem * d)],
                )

    return k(tT, tail)


@functools.partial(jax.jit, static_argnames=("hist",))
def _embed(table, ids_raw, hist):
    d = table.shape[1]
    hgs, bbs = ids_raw.shape[0], ids_raw.shape[1]
    dgs = d // SUB
    h2s = hist // 2

    mesh = plsc.VectorSubcoreMesh(core_axis_name="c", subcore_axis_name="s")

    @functools.partial(
        pl.kernel,
        out_type=jax.ShapeDtypeStruct((hist, dgs, bbs, SUB * LANE), jnp.float32),
        mesh=mesh,
        scratch_types=[
            pltpu.VMEM((hgs, SUB, LANE), jnp.int32),
            pltpu.VMEM((LANE, d), jnp.float32),
            pltpu.VMEM((LANE, d), jnp.float32),
            pltpu.VMEM((d * LANE,), jnp.float32),
            pltpu.VMEM((d * LANE,), jnp.float32),
            pltpu.SemaphoreType.DMA,
            pltpu.SemaphoreType.DMA,
            pltpu.SemaphoreType.DMA,
            pltpu.SemaphoreType.DMA,
        ],
        compiler_params=pltpu.CompilerParams(use_tc_tiling_on_sc=False, needs_layout_passes=False),
    )
    def k(table_hbm, ids_hbm, out_hbm, idx_v, rows0, rows1, t0, t1, sg0, sg1, ss0, ss1):
        wid = lax.axis_index("s") * NUM_CORES + lax.axis_index("c")
        rows = (rows0, rows1)
        tiles = (t0, t1)
        sg = (sg0, sg1)
        ss = (ss0, ss1)

        for hg in range(hgs):
            pltpu.sync_copy(ids_hbm.at[hg, wid], idx_v.at[hg])

        iota = lax.iota(jnp.int32, 16)

        def fire_gather(h, p):
            pltpu.async_copy(
                table_hbm.at[idx_v.at[h >> 3, h & 7]], rows[p], sg[p]
            )

        def wait_gather(p):
            pltpu.make_async_copy(table_hbm.at[idx_v.at[0, 0]], rows[p], sg[p]).wait()

        def fire_stores(h, p):
            for dg in range(dgs):
                pltpu.async_copy(
                    tiles[p].at[pl.ds(SUB * LANE * dg, SUB * LANE)],
                    out_hbm.at[h, dg, wid],
                    ss[p],
                )

        def wait_stores(h, p):
            for dg in range(dgs):
                pltpu.make_async_copy(
                    tiles[p].at[pl.ds(SUB * LANE * dg, SUB * LANE)],
                    out_hbm.at[h, dg, wid],
                    ss[p],
                ).wait()

        fire_gather(0, 0)
        fire_gather(1, 1)

        def body(h2, carry):
            for p in range(2):
                h = 2 * h2 + p
                q = p
                wait_gather(q)

                @pl.when(h2 > 0)
                def _():
                    wait_stores(h - 2, p)

                rp, tp = rows[q], tiles[p]

                def tbody(sv, c):
                    for su in range(4):
                        s = 4 * sv + su
                        dvec = (iota + s) & 15
                        dsti = dvec * LANE + iota
                        for d0 in range(0, d, 16):
                            dv = dvec + d0 if d0 else dvec
                            for lg in range(LANE // 16):
                                bv = iota + 16 * lg if lg else iota
                                dst = dsti + (d0 * LANE + 16 * lg) if d0 or lg else dsti
                                v = plsc.load_gather(rp, [bv, dv])
                                plsc.store_scatter(tp, [dst], v)
                    return c

                lax.fori_loop(0, 4, tbody, 0)
                fire_stores(h, p)

                @pl.when(h + 2 < hist)
                def _():
                    fire_gather(h + 2, q)
            return carry

        lax.fori_loop(0, hist // 2, body, 0)
        wait_stores(hist - 2, 0)
        wait_stores(hist - 1, 1)

    return k(table, ids_raw)


def kernel(input_ids, embedding_table):
    bsz, hist = input_ids.shape
    d = embedding_table.shape[1]
    bbs, hgs = bsz // LANE, hist // SUB
    ids_raw = (
        input_ids.astype(jnp.int32)
        .reshape(bbs, LANE, hgs, SUB)
        .transpose(2, 0, 3, 1)
    )
    vocab = embedding_table.shape[0]
    rem = vocab % LANE
    tail = embedding_table[vocab - rem:].reshape(-1)
    table_rm = _detile(embedding_table.T, tail).reshape(embedding_table.shape)
    out4d = _embed(table_rm, ids_raw, hist)
    out5d = out4d.reshape(hist, d // SUB, bbs, SUB, LANE)
    return out5d.transpose(2, 4, 0, 1, 3).reshape(bsz, hist, d)


# speedup vs baseline: 1.1839x; 1.0033x over previous
import functools

import jax
import jax.numpy as jnp
from jax import lax
from jax.experimental import pallas as pl
from jax.experimental.pallas import tpu as pltpu
from jax.experimental.pallas import tpu_sc as plsc

NUM_WORKERS = 32
NUM_CORES = 2
LANE = 128
SUB = 8


@jax.jit
def _detile(tT, tail):
    d, v = tT.shape
    ngrp = (v + 127) // 128
    full = v // 128
    rem = v - full * 128
    dgs = d // SUB
    words = 128 * d
    per_w = (full + NUM_WORKERS - 1) // NUM_WORKERS
    n2 = (per_w + (per_w & 1)) // 2

    mesh = plsc.VectorSubcoreMesh(core_axis_name="c", subcore_axis_name="s")

    @functools.partial(
        pl.kernel,
        out_type=jax.ShapeDtypeStruct((v * d,), jnp.float32),
        mesh=mesh,
        scratch_types=[
            pltpu.VMEM((d, 128), jnp.float32),
            pltpu.VMEM((d, 128), jnp.float32),
            pltpu.VMEM((words,), jnp.float32),
            pltpu.VMEM((words,), jnp.float32),
            pltpu.SemaphoreType.DMA,
            pltpu.SemaphoreType.DMA,
            pltpu.SemaphoreType.DMA,
            pltpu.SemaphoreType.DMA,
        ],
        compiler_params=pltpu.CompilerParams(
            use_tc_tiling_on_sc=True, needs_layout_passes=False
        ),
    )
    def k(tT_hbm, tail_hbm, out_hbm, in0, in1, ob0, ob1, si0, si1, so0, so1):
        wid = lax.axis_index("s") * NUM_CORES + lax.axis_index("c")
        inb = (in0, in1)
        outb = (ob0, ob1)
        si = (si0, si1)
        so = (so0, so1)

        iota = lax.iota(jnp.int32, 16)

        def cidx(i):
            return jnp.minimum(wid * per_w + i, full - 1)

        def fire_in(i, p):
            c = cidx(i)
            pltpu.async_copy(
                tT_hbm.at[:, pl.ds(128 * c, 128)], inb[p], si[p]
            )

        def wait_in(p):
            pltpu.make_async_copy(
                tT_hbm.at[:, pl.ds(0, 128)], inb[p], si[p]
            ).wait()

        def fire_out(i, p):
            pltpu.async_copy(outb[p], out_hbm.at[pl.ds(words * cidx(i), words)], so[p])

        def wait_out(i, p):
            pltpu.make_async_copy(
                outb[p], out_hbm.at[pl.ds(words * cidx(i), words)], so[p]
            ).wait()

        def transpose(p, lgroups):
            ip, op = inb[p], outb[p]

            def tbody(sv, carry):
                for su in range(4):
                    s = 4 * sv + su
                    lv0 = (iota + s) & 15
                    lvd = lv0 * d + iota
                    for dg0 in range(dgs // 2):
                        base = iota + 16 * dg0 if dg0 else iota
                        dst0 = lvd + 16 * dg0 if dg0 else lvd
                        for l0 in range(0, 16 * lgroups, 16):
                            lv = lv0 + l0 if l0 else lv0
                            dst = dst0 + l0 * d if l0 else dst0
                            val = plsc.load_gather(ip, [base, lv])
                            plsc.store_scatter(op, [dst], val)
                return carry

            lax.fori_loop(0, 4, tbody, 0)

        fire_in(0, 0)
        fire_in(1, 1)

        def body(i2, carry):
            for p in range(2):
                i = 2 * i2 + p
                wait_in(p)

                @pl.when(i2 > 0)
                def _():
                    wait_out(i - 2, p)

                transpose(p, 8)
                fire_out(i, p)

                @pl.when(i + 2 < 2 * n2)
                def _():
                    fire_in(i + 2, p)
            return carry

        lax.fori_loop(0, n2, body, 0)
        wait_out(2 * n2 - 2, 0)
        wait_out(2 * n2 - 1, 1)

        if rem:
            @pl.when(wid == NUM_WORKERS - 1)
            def _():
                pltpu.sync_copy(tail_hbm, outb[0].at[pl.ds(0, rem * d)])
                pltpu.sync_copy(
                    outb[0].at[pl.ds(0, rem * d)],
                    out_hbm.at[pl.ds(words * full, rem * d)],
                )

    return k(tT, tail)


@functools.partial(jax.jit, static_argnames=("hist",))
def _embed(table, ids_raw, hist):
    d = table.shape[1]
    hgs, bbs = ids_raw.shape[0], ids_raw.shape[1]
    dgs = d // SUB
    h2s = hist // 2

    mesh = plsc.VectorSubcoreMesh(core_axis_name="c", subcore_axis_name="s")

    @functools.partial(
        pl.kernel,
        out_type=jax.ShapeDtypeStruct((hist, dgs, bbs, SUB * LANE), jnp.float32),
        mesh=mesh,
        scratch_types=[
            pltpu.VMEM((hgs, SUB, LANE), jnp.int32),
            pltpu.VMEM((LANE, d), jnp.float32),
            pltpu.VMEM((LANE, d), jnp.float32),
            pltpu.VMEM((d * LANE,), jnp.float32),
            pltpu.VMEM((d * LANE,), jnp.float32),
            pltpu.SemaphoreType.DMA,
            pltpu.SemaphoreType.DMA,
            pltpu.SemaphoreType.DMA,
            pltpu.SemaphoreType.DMA,
        ],
        compiler_params=pltpu.CompilerParams(use_tc_tiling_on_sc=False, needs_layout_passes=False),
    )
    def k(table_hbm, ids_hbm, out_hbm, idx_v, rows0, rows1, t0, t1, sg0, sg1, ss0, ss1):
        wid = lax.axis_index("s") * NUM_CORES + lax.axis_index("c")
        rows = (rows0, rows1)
        tiles = (t0, t1)
        sg = (sg0, sg1)
        ss = (ss0, ss1)

        for hg in range(hgs):
            pltpu.sync_copy(ids_hbm.at[hg, wid], idx_v.at[hg])

        iota = lax.iota(jnp.int32, 16)

        def fire_gather(h, p):
            pltpu.async_copy(
                table_hbm.at[idx_v.at[h >> 3, h & 7]], rows[p], sg[p]
            )

        def wait_gather(p):
            pltpu.make_async_copy(table_hbm.at[idx_v.at[0, 0]], rows[p], sg[p]).wait()

        def fire_stores(h, p):
            for dg in range(dgs):
                pltpu.async_copy(
                    tiles[p].at[pl.ds(SUB * LANE * dg, SUB * LANE)],
                    out_hbm.at[h, dg, wid],
                    ss[p],
                )

        def wait_stores(h, p):
            for dg in range(dgs):
                pltpu.make_async_copy(
                    tiles[p].at[pl.ds(SUB * LANE * dg, SUB * LANE)],
                    out_hbm.at[h, dg, wid],
                    ss[p],
                ).wait()

        fire_gather(0, 0)
        fire_gather(1, 1)

        def body(h2, carry):
            for p in range(2):
                h = 2 * h2 + p
                q = p
                wait_gather(q)

                @pl.when(h2 > 0)
                def _():
                    wait_stores(h - 2, p)

                rp, tp = rows[q], tiles[p]

                def tbody(sv, c):
                    for su in range(4):
                        s = 4 * sv + su
                        dvec = (iota + s) & 15
                        dsti = dvec * LANE + iota
                        for d0 in range(0, d, 16):
                            dv = dvec + d0 if d0 else dvec
                            for lg in range(LANE // 16):
                                bv = iota + 16 * lg if lg else iota
                                dst = dsti + (d0 * LANE + 16 * lg) if d0 or lg else dsti
                                v = plsc.load_gather(rp, [bv, dv])
                                plsc.store_scatter(tp, [dst], v)
                    return c

                lax.fori_loop(0, 4, tbody, 0)
                fire_stores(h, p)

                @pl.when(h + 2 < hist)
                def _():
                    fire_gather(h + 2, q)
            return carry

        lax.fori_loop(0, hist // 2, body, 0)
        wait_stores(hist - 2, 0)
        wait_stores(hist - 1, 1)

    return k(table, ids_raw)


def kernel(input_ids, embedding_table):
    bsz, hist = input_ids.shape
    d = embedding_table.shape[1]
    bbs, hgs = bsz // LANE, hist // SUB
    ids_raw = (
        input_ids.astype(jnp.int32)
        .reshape(bbs, LANE, hgs, SUB)
        .transpose(2, 0, 3, 1)
    )
    vocab = embedding_table.shape[0]
    rem = vocab % LANE
    tail = embedding_table[vocab - rem:].reshape(-1)
    table_rm = _detile(embedding_table.T, tail).reshape(embedding_table.shape)
    out4d = _embed(table_rm, ids_raw, hist)
    out5d = out4d.reshape(hist, d // SUB, bbs, SUB, LANE)
    return out5d.transpose(2, 4, 0, 1, 3).reshape(bsz, hist, d)


# speedup vs baseline: 1.1854x; 1.0013x over previous
import functools

import jax
import jax.numpy as jnp
from jax import lax
from jax.experimental import pallas as pl
from jax.experimental.pallas import tpu as pltpu
from jax.experimental.pallas import tpu_sc as plsc

NUM_WORKERS = 32
NUM_CORES = 2
LANE = 128
SUB = 8


@jax.jit
def _detile(tT, tail):
    d, v = tT.shape
    full = v // 128
    rem = v - full * 128
    dgs = d // SUB
    words = 128 * d
    per_w = (full + NUM_WORKERS - 1) // NUM_WORKERS
    n2 = (per_w + (per_w & 1)) // 2

    mesh = plsc.VectorSubcoreMesh(core_axis_name="c", subcore_axis_name="s")

    @functools.partial(
        pl.kernel,
        out_type=jax.ShapeDtypeStruct((v * d,), jnp.float32),
        mesh=mesh,
        scratch_types=[
            pltpu.VMEM((d, 128), jnp.float32),
            pltpu.VMEM((d, 128), jnp.float32),
            pltpu.VMEM((words,), jnp.float32),
            pltpu.VMEM((words,), jnp.float32),
            pltpu.SemaphoreType.DMA,
            pltpu.SemaphoreType.DMA,
            pltpu.SemaphoreType.DMA,
            pltpu.SemaphoreType.DMA,
        ],
        compiler_params=pltpu.CompilerParams(
            use_tc_tiling_on_sc=True, needs_layout_passes=False
        ),
    )
    def k(tT_hbm, tail_hbm, out_hbm, in0, in1, ob0, ob1, si0, si1, so0, so1):
        wid = lax.axis_index("s") * NUM_CORES + lax.axis_index("c")
        inb = (in0, in1)
        outb = (ob0, ob1)
        si = (si0, si1)
        so = (so0, so1)

        iota = lax.iota(jnp.int32, 16)

        def cidx(i):
            return jnp.minimum(wid * per_w + i, full - 1)

        def fire_in(i, p):
            c = cidx(i)
            pltpu.async_copy(
                tT_hbm.at[:, pl.ds(128 * c, 128)], inb[p], si[p]
            )

        def wait_in(p):
            pltpu.make_async_copy(
                tT_hbm.at[:, pl.ds(0, 128)], inb[p], si[p]
            ).wait()

        def fire_out(i, p):
            pltpu.async_copy(outb[p], out_hbm.at[pl.ds(words * cidx(i), words)], so[p])

        def wait_out(i, p):
            pltpu.make_async_copy(
                outb[p], out_hbm.at[pl.ds(words * cidx(i), words)], so[p]
            ).wait()

        def transpose(p, lgroups):
            ip, op = inb[p], outb[p]

            def tbody(sv, carry):
                for su in range(4):
                    s = 4 * sv + su
                    lv0 = (iota + s) & 15
                    lvd = lv0 * d + iota
                    for dg0 in range(dgs // 2):
                        base = iota + 16 * dg0 if dg0 else iota
                        dst0 = lvd + 16 * dg0 if dg0 else lvd
                        for l0 in range(0, 16 * lgroups, 16):
                            lv = lv0 + l0 if l0 else lv0
                            dst = dst0 + l0 * d if l0 else dst0
                            val = plsc.load_gather(ip, [base, lv])
                            plsc.store_scatter(op, [dst], val)
                return carry

            lax.fori_loop(0, 4, tbody, 0)

        fire_in(0, 0)
        fire_in(1, 1)

        def body(i2, carry):
            for p in range(2):
                i = 2 * i2 + p
                wait_in(p)

                @pl.when(i2 > 0)
                def _():
                    wait_out(i - 2, p)

                transpose(p, 8)
                fire_out(i, p)

                @pl.when(i + 2 < 2 * n2)
                def _():
                    fire_in(i + 2, p)
            return carry

        lax.fori_loop(0, n2, body, 0)
        wait_out(2 * n2 - 2, 0)
        wait_out(2 * n2 - 1, 1)

        if rem:
            @pl.when(wid == NUM_WORKERS - 1)
            def _():
                pltpu.sync_copy(tail_hbm, outb[0].at[pl.ds(0, rem * d)])
                pltpu.sync_copy(
                    outb[0].at[pl.ds(0, rem * d)],
                    out_hbm.at[pl.ds(words * full, rem * d)],
                )

    return k(tT, tail)


@functools.partial(jax.jit, static_argnames=("hist",))
def _embed(table, ids_raw, hist):
    d = table.shape[1]
    hgs, bbs = ids_raw.shape[0], ids_raw.shape[1]
    dgs = d // SUB

    mesh = plsc.VectorSubcoreMesh(core_axis_name="c", subcore_axis_name="s")

    @functools.partial(
        pl.kernel,
        out_type=jax.ShapeDtypeStruct((hist, dgs, bbs, SUB * LANE), jnp.float32),
        mesh=mesh,
        scratch_types=[
            pltpu.VMEM((hgs, SUB, LANE), jnp.int32),
            pltpu.VMEM((LANE, d), jnp.float32),
            pltpu.VMEM((LANE, d), jnp.float32),
            pltpu.VMEM((d * LANE,), jnp.float32),
            pltpu.VMEM((d * LANE,), jnp.float32),
            pltpu.SemaphoreType.DMA,
            pltpu.SemaphoreType.DMA,
            pltpu.SemaphoreType.DMA,
            pltpu.SemaphoreType.DMA,
        ],
        compiler_params=pltpu.CompilerParams(use_tc_tiling_on_sc=False, needs_layout_passes=False),
    )
    def k(table_hbm, ids_hbm, out_hbm, idx_v, rows0, rows1, t0, t1, sg0, sg1, ss0, ss1):
        wid = lax.axis_index("s") * NUM_CORES + lax.axis_index("c")
        rows = (rows0, rows1)
        tiles = (t0, t1)
        sg = (sg0, sg1)
        ss = (ss0, ss1)

        for hg in range(hgs):
            pltpu.sync_copy(ids_hbm.at[hg, wid], idx_v.at[hg])

        iota = lax.iota(jnp.int32, 16)

        def fire_gather(h, p):
            pltpu.async_copy(
                table_hbm.at[idx_v.at[h >> 3, h & 7]], rows[p], sg[p]
            )

        def wait_gather(p):
            pltpu.make_async_copy(table_hbm.at[idx_v.at[0, 0]], rows[p], sg[p]).wait()

        def fire_stores(h, p):
            for dg in range(dgs):
                pltpu.async_copy(
                    tiles[p].at[pl.ds(SUB * LANE * dg, SUB * LANE)],
                    out_hbm.at[h, dg, wid],
                    ss[p],
                )

        def wait_stores(h, p):
            for dg in range(dgs):
                pltpu.make_async_copy(
                    tiles[p].at[pl.ds(SUB * LANE * dg, SUB * LANE)],
                    out_hbm.at[h, dg, wid],
                    ss[p],
                ).wait()

        fire_gather(0, 0)
        fire_gather(1, 1)

        def body(h2, carry):
            for p in range(2):
                h = 2 * h2 + p
                wait_gather(p)

                @pl.when(h2 > 0)
                def _():
                    wait_stores(h - 2, p)

                rp, tp = rows[p], tiles[p]

                def tbody(sv, c):
                    for su in range(4):
                        s = 4 * sv + su
                        dvec = (iota + s) & 15
                        dsti = dvec * LANE + iota
                        for d0 in range(0, d, 16):
                            dv = dvec + d0 if d0 else dvec
                            for lg in range(LANE // 16):
                                bv = iota + 16 * lg if lg else iota
                                dst = dsti + (d0 * LANE + 16 * lg) if d0 or lg else dsti
                                v = plsc.load_gather(rp, [bv, dv])
                                plsc.store_scatter(tp, [dst], v)
                    return c

                lax.fori_loop(0, 4, tbody, 0)
                fire_stores(h, p)

                @pl.when(h + 2 < hist)
                def _():
                    fire_gather(h + 2, p)
            return carry

        lax.fori_loop(0, hist // 2, body, 0)
        wait_stores(hist - 2, 0)
        wait_stores(hist - 1, 1)

    return k(table, ids_raw)


def kernel(input_ids, embedding_table):
    bsz, hist = input_ids.shape
    d = embedding_table.shape[1]
    bbs, hgs = bsz // LANE, hist // SUB
    ids_raw = (
        input_ids.astype(jnp.int32)
        .reshape(bbs, LANE, hgs, SUB)
        .transpose(2, 0, 3, 1)
    )
    vocab = embedding_table.shape[0]
    rem = vocab % LANE
    tail = embedding_table[vocab - rem:].reshape(-1)
    table_rm = _detile(embedding_table.T, tail).reshape(embedding_table.shape)
    out4d = _embed(table_rm, ids_raw, hist)
    out5d = out4d.reshape(hist, d // SUB, bbs, SUB, LANE)
    return out5d.transpose(2, 4, 0, 1, 3).reshape(bsz, hist, d)
